# baseline (speedup 1.0000x reference)
import functools

import jax
import jax.numpy as jnp
from jax.experimental import pallas as pl
from jax.experimental.pallas import tpu as pltpu

N = 10000
K = 32
D = 128
BN = 200


def _body(self_ref, nf_ref, w_ref, b_ref, g_ref, beta_ref, out_ref):
    agg = jnp.sum(nf_ref[...], axis=0)
    x = jnp.concatenate([self_ref[...], agg], axis=1)
    out = jax.lax.dot_general(
        x, w_ref[...], (((1,), (0,)), ((), ())),
        preferred_element_type=jnp.float32,
    ) + b_ref[...]
    mu = jnp.mean(out, axis=-1, keepdims=True)
    var = jnp.mean(jnp.square(out - mu), axis=-1, keepdims=True)
    normed = (out - mu) * jax.lax.rsqrt(var + 1e-5) * g_ref[...] + beta_ref[...]
    out_ref[...] = jnp.maximum(normed, 0.0)


@jax.jit
def kernel(self_feat, neighbor_feats, W_self, b_self, W_nb, b_nb, ln_gamma, ln_beta):
    w_cat = jnp.concatenate([W_self.T, W_nb.T / K], axis=0)
    bias = (b_self + b_nb).reshape(1, D)
    gamma = ln_gamma.reshape(1, D)
    beta = ln_beta.reshape(1, D)

    grid = (N // BN,)
    return pl.pallas_call(
        _body,
        grid=grid,
        in_specs=[
            pl.BlockSpec((BN, D), lambda i: (i, 0)),
            pl.BlockSpec((K, BN, D), lambda i: (0, i, 0)),
            pl.BlockSpec((2 * D, D), lambda i: (0, 0)),
            pl.BlockSpec((1, D), lambda i: (0, 0)),
            pl.BlockSpec((1, D), lambda i: (0, 0)),
            pl.BlockSpec((1, D), lambda i: (0, 0)),
        ],
        out_specs=pl.BlockSpec((BN, D), lambda i: (i, 0)),
        out_shape=jax.ShapeDtypeStruct((N, D), jnp.float32),
        compiler_params=pltpu.CompilerParams(
            dimension_semantics=("arbitrary",),
        ),
    )(self_feat, neighbor_feats, w_cat, bias, gamma, beta)


# speedup vs baseline: 1.2326x; 1.2326x over previous
import functools

import jax
import jax.numpy as jnp
from jax.experimental import pallas as pl
from jax.experimental.pallas import tpu as pltpu

N = 10000
K = 32
D = 128
BN = 640


def _body(self_ref, nf_ref, w_ref, b_ref, g_ref, beta_ref, out_ref):
    agg = jnp.sum(nf_ref[...], axis=0)
    x = jnp.concatenate([self_ref[...], agg], axis=1)
    out = jax.lax.dot_general(
        x, w_ref[...], (((1,), (0,)), ((), ())),
        preferred_element_type=jnp.float32,
    ) + b_ref[...]
    mu = jnp.mean(out, axis=-1, keepdims=True)
    var = jnp.mean(jnp.square(out - mu), axis=-1, keepdims=True)
    normed = (out - mu) * jax.lax.rsqrt(var + 1e-5) * g_ref[...] + beta_ref[...]
    out_ref[...] = jnp.maximum(normed, 0.0)


@jax.jit
def kernel(self_feat, neighbor_feats, W_self, b_self, W_nb, b_nb, ln_gamma, ln_beta):
    w_cat = jnp.concatenate([W_self.T, W_nb.T / K], axis=0)
    bias = (b_self + b_nb).reshape(1, D)
    gamma = ln_gamma.reshape(1, D)
    beta = ln_beta.reshape(1, D)

    grid = (pl.cdiv(N, BN),)
    return pl.pallas_call(
        _body,
        grid=grid,
        in_specs=[
            pl.BlockSpec((BN, D), lambda i: (i, 0)),
            pl.BlockSpec((K, BN, D), lambda i: (0, i, 0)),
            pl.BlockSpec((2 * D, D), lambda i: (0, 0)),
            pl.BlockSpec((1, D), lambda i: (0, 0)),
            pl.BlockSpec((1, D), lambda i: (0, 0)),
            pl.BlockSpec((1, D), lambda i: (0, 0)),
        ],
        out_specs=pl.BlockSpec((BN, D), lambda i: (i, 0)),
        out_shape=jax.ShapeDtypeStruct((N, D), jnp.float32),
        compiler_params=pltpu.CompilerParams(
            dimension_semantics=("arbitrary",),
        ),
    )(self_feat, neighbor_feats, w_cat, bias, gamma, beta)


# speedup vs baseline: 1.2555x; 1.0186x over previous
import functools

import jax
import jax.numpy as jnp
from jax.experimental import pallas as pl
from jax.experimental.pallas import tpu as pltpu

N = 10000
K = 32
D = 128
BN = 400


def _body(self_ref, nf_ref, w_ref, b_ref, g_ref, beta_ref, out_ref):
    agg = jnp.sum(nf_ref[...], axis=0)
    x = jnp.concatenate([self_ref[...], agg], axis=1)
    out = jax.lax.dot_general(
        x, w_ref[...], (((1,), (0,)), ((), ())),
        preferred_element_type=jnp.float32,
    ) + b_ref[...]
    mu = jnp.mean(out, axis=-1, keepdims=True)
    var = jnp.mean(jnp.square(out - mu), axis=-1, keepdims=True)
    normed = (out - mu) * jax.lax.rsqrt(var + 1e-5) * g_ref[...] + beta_ref[...]
    out_ref[...] = jnp.maximum(normed, 0.0)


@jax.jit
def kernel(self_feat, neighbor_feats, W_self, b_self, W_nb, b_nb, ln_gamma, ln_beta):
    w_cat = jnp.concatenate([W_self.T, W_nb.T / K], axis=0)
    bias = (b_self + b_nb).reshape(1, D)
    gamma = ln_gamma.reshape(1, D)
    beta = ln_beta.reshape(1, D)

    grid = (pl.cdiv(N, BN),)
    return pl.pallas_call(
        _body,
        grid=grid,
        in_specs=[
            pl.BlockSpec((BN, D), lambda i: (i, 0)),
            pl.BlockSpec((K, BN, D), lambda i: (0, i, 0)),
            pl.BlockSpec((2 * D, D), lambda i: (0, 0)),
            pl.BlockSpec((1, D), lambda i: (0, 0)),
            pl.BlockSpec((1, D), lambda i: (0, 0)),
            pl.BlockSpec((1, D), lambda i: (0, 0)),
        ],
        out_specs=pl.BlockSpec((BN, D), lambda i: (i, 0)),
        out_shape=jax.ShapeDtypeStruct((N, D), jnp.float32),
        compiler_params=pltpu.CompilerParams(
            dimension_semantics=("arbitrary",),
        ),
    )(self_feat, neighbor_feats, w_cat, bias, gamma, beta)


# speedup vs baseline: 1.2557x; 1.0002x over previous
import jax
import jax.numpy as jnp
from jax.experimental import pallas as pl
from jax.experimental.pallas import tpu as pltpu

N = 10000
K = 32
D = 128
BN = 400
NS = 2


def _body(self_ref, *rest):
    nf_refs = rest[:NS]
    w_ref, b_ref, g_ref, beta_ref, out_ref = rest[NS:]
    agg = jnp.sum(nf_refs[0][...], axis=0)
    for r in nf_refs[1:]:
        agg = agg + jnp.sum(r[...], axis=0)
    x = jnp.concatenate([self_ref[...], agg], axis=1)
    out = jax.lax.dot_general(
        x, w_ref[...], (((1,), (0,)), ((), ())),
        preferred_element_type=jnp.float32,
    ) + b_ref[...]
    mu = jnp.mean(out, axis=-1, keepdims=True)
    var = jnp.mean(jnp.square(out - mu), axis=-1, keepdims=True)
    normed = (out - mu) * jax.lax.rsqrt(var + 1e-5) * g_ref[...] + beta_ref[...]
    out_ref[...] = jnp.maximum(normed, 0.0)


@jax.jit
def kernel(self_feat, neighbor_feats, W_self, b_self, W_nb, b_nb, ln_gamma, ln_beta):
    w_cat = jnp.concatenate([W_self.T, W_nb.T / K], axis=0)
    bias = (b_self + b_nb).reshape(1, D)
    gamma = ln_gamma.reshape(1, D)
    beta = ln_beta.reshape(1, D)

    grid = (pl.cdiv(N, BN),)
    nf_specs = [
        pl.BlockSpec((K // NS, BN, D), lambda i, s=s: (s, i, 0))
        for s in range(NS)
    ]
    return pl.pallas_call(
        _body,
        grid=grid,
        in_specs=[
            pl.BlockSpec((BN, D), lambda i: (i, 0)),
            *nf_specs,
            pl.BlockSpec((2 * D, D), lambda i: (0, 0)),
            pl.BlockSpec((1, D), lambda i: (0, 0)),
            pl.BlockSpec((1, D), lambda i: (0, 0)),
            pl.BlockSpec((1, D), lambda i: (0, 0)),
        ],
        out_specs=pl.BlockSpec((BN, D), lambda i: (i, 0)),
        out_shape=jax.ShapeDtypeStruct((N, D), jnp.float32),
        compiler_params=pltpu.CompilerParams(
            dimension_semantics=("arbitrary",),
        ),
    )(self_feat, *([neighbor_feats] * NS), w_cat, bias, gamma, beta)
